# baseline (speedup 1.0000x reference)
import functools

import jax
import jax.numpy as jnp
from jax import lax
from jax.experimental import pallas as pl
from jax.experimental.pallas import tpu as pltpu
from jax.experimental.pallas import tpu_sc as plsc

N = 10000
E = 320000
DF = 128
NHEADS = 8
NHID = 16
TW = 144
ALPHA = 0.2
EPS = 1e-16

NC = 2
NS = 16
EPT = E // (NC * NS)
CHUNK = 40
NCHUNK = EPT // CHUNK
IBLK = 50
NIBLK = NCHUNK // IBLK
NPAD = 10240
ZROWS = NPAD // NS

ROWBLK = 1000
NBLK = N // ROWBLK


def _pre_body(x_ref, wall_ref, a1_ref, a2_ref, pmat_ref, tb_ref, s1_ref,
              s2_ref):
    h = jnp.dot(x_ref[...], wall_ref[...], preferred_element_type=jnp.float32)
    s1_ref[...] = jnp.dot(h, a1_ref[...], preferred_element_type=jnp.float32)
    s2_ref[...] = jnp.dot(h, a2_ref[...], preferred_element_type=jnp.float32)
    hp = jnp.dot(h, pmat_ref[...], preferred_element_type=jnp.float32)
    tb_ref[...] = hp.astype(jnp.bfloat16)


_pre_call = pl.pallas_call(
    _pre_body,
    grid=(NBLK,),
    in_specs=[
        pl.BlockSpec((ROWBLK, DF), lambda i: (i, 0)),
        pl.BlockSpec((DF, DF), lambda i: (0, 0)),
        pl.BlockSpec((DF, 16), lambda i: (0, 0)),
        pl.BlockSpec((DF, 16), lambda i: (0, 0)),
        pl.BlockSpec((DF, DF), lambda i: (0, 0)),
    ],
    out_specs=[
        pl.BlockSpec((ROWBLK, DF), lambda i: (i, 0)),
        pl.BlockSpec((ROWBLK, 16), lambda i: (i, 0)),
        pl.BlockSpec((ROWBLK, 16), lambda i: (i, 0)),
    ],
    out_shape=[
        jax.ShapeDtypeStruct((N, DF), jnp.bfloat16),
        jax.ShapeDtypeStruct((N, 16), jnp.float32),
        jax.ShapeDtypeStruct((N, 16), jnp.float32),
    ],
)


def _mid_body(acc_ref, wout_ref, a1_ref, a2_ref, pmat_ref, tb_ref, s1_ref,
              s2_ref):
    s = acc_ref[0] + acc_ref[1]
    cols = []
    for i in range(NHEADS):
        hi = s[:, NHID * i:NHID * (i + 1)]
        ri = s[:, DF + i:DF + i + 1]
        cols.append(hi / (ri + EPS))
    x2 = jnp.concatenate(cols, axis=1)
    x2 = jnp.where(x2 > 0, x2, jnp.exp(x2) - 1.0)
    h = jnp.dot(x2, wout_ref[...], preferred_element_type=jnp.float32)
    s1_ref[...] = jnp.dot(h, a1_ref[...], preferred_element_type=jnp.float32)
    s2_ref[...] = jnp.dot(h, a2_ref[...], preferred_element_type=jnp.float32)
    hp = jnp.dot(h, pmat_ref[...], preferred_element_type=jnp.float32)
    tb_ref[...] = hp.astype(jnp.bfloat16)


_mid_call = pl.pallas_call(
    _mid_body,
    grid=(NBLK,),
    in_specs=[
        pl.BlockSpec((NC, ROWBLK, TW), lambda i: (0, i, 0)),
        pl.BlockSpec((DF, DF), lambda i: (0, 0)),
        pl.BlockSpec((DF, 16), lambda i: (0, 0)),
        pl.BlockSpec((DF, 16), lambda i: (0, 0)),
        pl.BlockSpec((DF, DF), lambda i: (0, 0)),
    ],
    out_specs=[
        pl.BlockSpec((ROWBLK, DF), lambda i: (i, 0)),
        pl.BlockSpec((ROWBLK, 16), lambda i: (i, 0)),
        pl.BlockSpec((ROWBLK, 16), lambda i: (i, 0)),
    ],
    out_shape=[
        jax.ShapeDtypeStruct((N, DF), jnp.bfloat16),
        jax.ShapeDtypeStruct((N, 16), jnp.float32),
        jax.ShapeDtypeStruct((N, 16), jnp.float32),
    ],
)


def _post_body(acc_ref, o_ref):
    s = acc_ref[0] + acc_ref[1]
    x = s[:, 0:DF] / (s[:, DF:DF + 1] + EPS)
    o_ref[...] = jnp.where(x > 0, x, jnp.exp(x) - 1.0)


_post_call = pl.pallas_call(
    _post_body,
    grid=(NBLK,),
    in_specs=[pl.BlockSpec((NC, ROWBLK, TW), lambda i: (0, i, 0))],
    out_specs=pl.BlockSpec((ROWBLK, DF), lambda i: (i, 0)),
    out_shape=jax.ShapeDtypeStruct((N, DF), jnp.float32),
)


def _lane_broadcast(v, j):
    idx = jnp.full((16, 1), j, dtype=jnp.int32)
    dn = lax.GatherDimensionNumbers(
        offset_dims=(), collapsed_slice_dims=(0,), start_index_map=(0,))
    return lax.gather(v, idx, dn, slice_sizes=(1,),
                      mode=lax.GatherScatterMode.PROMISE_IN_BOUNDS)


def _edge_pass_body(t_hbm, s1_hbm, s2_hbm, src_hbm, dst_hbm, zeros_hbm,
                    out_hbm, srcblk, dstblk, gbuf0, gbuf1, s1b0, s1b1,
                    s2b0, s2b1, mb0, mb1, accum, gsem0, gsem1, s1sem0,
                    s1sem1, s2sem0, s2sem1, ssem0, ssem1):
    cid = lax.axis_index("c")
    sid = lax.axis_index("s")
    crow = (cid * NS + sid) * NCHUNK
    zrow0 = pl.multiple_of(sid * ZROWS, ZROWS)

    gbufs = [gbuf0, gbuf1]
    s1bufs = [s1b0, s1b1]
    s2bufs = [s2b0, s2b1]
    mbufs = [mb0, mb1]
    gsems = [gsem0, gsem1]
    s1sems = [s1sem0, s1sem1]
    s2sems = [s2sem0, s2sem1]
    ssems = [ssem0, ssem1]

    pltpu.sync_copy(zeros_hbm, accum.at[pl.ds(zrow0, ZROWS)])
    plsc.subcore_barrier()

    def issue_gather(c, b):
        pltpu.async_copy(t_hbm.at[dstblk.at[c]], gbufs[b], gsems[b])
        pltpu.async_copy(s1_hbm.at[srcblk.at[c]], s1bufs[b], s1sems[b])
        pltpu.async_copy(s2_hbm.at[dstblk.at[c]], s2bufs[b], s2sems[b])

    def wait_gather(c, b):
        pltpu.make_async_copy(t_hbm.at[dstblk.at[c]], gbufs[b], gsems[b]).wait()
        pltpu.make_async_copy(s1_hbm.at[srcblk.at[c]], s1bufs[b],
                              s1sems[b]).wait()
        pltpu.make_async_copy(s2_hbm.at[dstblk.at[c]], s2bufs[b],
                              s2sems[b]).wait()

    def issue_scatter(c, b):
        pltpu.async_copy(mbufs[b], accum.at[srcblk.at[c]], ssems[b], add=True)

    def wait_scatter(c, b):
        pltpu.make_async_copy(mbufs[b], accum.at[srcblk.at[c]],
                              ssems[b]).wait()

    def compute(b):
        gbuf, s1buf, s2buf, mbuf = gbufs[b], s1bufs[b], s2bufs[b], mbufs[b]

        def edge_body(e):
            s1 = s1buf[e, :]
            s2 = s2buf[e, :]
            t = s1 + s2
            w = jnp.exp(-jnp.maximum(t, ALPHA * t))
            mbuf[e, pl.ds(DF, 16)] = w
            for g in range(NHEADS // 2):
                hb = gbuf[e, pl.ds(32 * g, 32)]
                ha, hc = plsc.unpack(hb, format=plsc.PackFormat.INTERLEAVED,
                                     preferred_element_type=jnp.float32)
                mbuf[e, pl.ds(32 * g, 16)] = _lane_broadcast(w, 2 * g) * ha
                mbuf[e, pl.ds(32 * g + 16, 16)] = (
                    _lane_broadcast(w, 2 * g + 1) * hc)

        plsc.parallel_loop(0, CHUNK, 1, unroll=4)(edge_body)

    def block_body(blk, carry):
        brow = crow + blk * IBLK
        pltpu.sync_copy(src_hbm.at[pl.ds(brow, IBLK)], srcblk)
        pltpu.sync_copy(dst_hbm.at[pl.ds(brow, IBLK)], dstblk)
        issue_gather(0, 0)
        issue_gather(1, 1)

        def pair_body(p, carry2):
            for b in range(2):
                cb = 2 * p + b
                wait_gather(cb, b)

                @pl.when(cb >= 2)
                def _():
                    wait_scatter(cb - 2, b)

                compute(b)
                issue_scatter(cb, b)
                issue_gather(cb + 2, b)
            return carry2

        lax.fori_loop(0, IBLK // 2 - 1, pair_body, 0)

        for cb in (IBLK - 2, IBLK - 1):
            b = cb % 2
            wait_gather(cb, b)
            wait_scatter(cb - 2, b)
            compute(b)
            issue_scatter(cb, b)
        wait_scatter(IBLK - 2, (IBLK - 2) % 2)
        wait_scatter(IBLK - 1, (IBLK - 1) % 2)
        return carry

    lax.fori_loop(0, NIBLK, block_body, 0)

    plsc.subcore_barrier()
    pltpu.sync_copy(accum.at[pl.ds(zrow0, ZROWS)],
                    out_hbm.at[cid, pl.ds(zrow0, ZROWS)])


_EDGE_PASS_CACHE = []


def _edge_pass(*args):
    if not _EDGE_PASS_CACHE:
        mesh = plsc.VectorSubcoreMesh(core_axis_name="c", subcore_axis_name="s")
        _EDGE_PASS_CACHE.append(functools.partial(
            pl.kernel,
            out_type=jax.ShapeDtypeStruct((NC, NPAD, TW), jnp.float32),
            mesh=mesh,
            scratch_types=[
                pltpu.VMEM((IBLK, CHUNK), jnp.int32),
                pltpu.VMEM((IBLK, CHUNK), jnp.int32),
                pltpu.VMEM((CHUNK, DF), jnp.bfloat16),
                pltpu.VMEM((CHUNK, DF), jnp.bfloat16),
                pltpu.VMEM((CHUNK, 16), jnp.float32),
                pltpu.VMEM((CHUNK, 16), jnp.float32),
                pltpu.VMEM((CHUNK, 16), jnp.float32),
                pltpu.VMEM((CHUNK, 16), jnp.float32),
                pltpu.VMEM((CHUNK, TW), jnp.float32),
                pltpu.VMEM((CHUNK, TW), jnp.float32),
                pltpu.VMEM_SHARED((NPAD, TW), jnp.float32),
                pltpu.SemaphoreType.DMA,
                pltpu.SemaphoreType.DMA,
                pltpu.SemaphoreType.DMA,
                pltpu.SemaphoreType.DMA,
                pltpu.SemaphoreType.DMA,
                pltpu.SemaphoreType.DMA,
                pltpu.SemaphoreType.DMA,
                pltpu.SemaphoreType.DMA,
            ],
            compiler_params=pltpu.CompilerParams(
                use_tc_tiling_on_sc=False, needs_layout_passes=False),
        )(_edge_pass_body))
    return _EDGE_PASS_CACHE[0](*args)


def kernel(Corpus_, batch_inputs, entity_embeddings, edge_list, W, a, W_out, a_out):
    x = entity_embeddings
    src = edge_list[0].reshape(E // CHUNK, CHUNK)
    dst = edge_list[1].reshape(E // CHUNK, CHUNK)

    wall = jnp.transpose(W, (1, 0, 2)).reshape(DF, DF)
    a1 = a[:, 0, :NHID]
    a2 = a[:, 0, NHID:]
    heads = jnp.arange(NHEADS)
    rows = (NHID * heads[:, None] + jnp.arange(NHID)[None, :]).reshape(-1)
    cols = jnp.repeat(heads, NHID)
    A1p = jnp.zeros((DF, 16), jnp.float32).at[rows, cols].set(a1.reshape(-1))
    A2p = jnp.zeros((DF, 16), jnp.float32).at[rows, cols].set(a2.reshape(-1))

    ao1 = a_out[0, :DF]
    ao2 = a_out[0, DF:]
    A1o = jnp.tile(ao1[:, None], (1, 16))
    A2o = jnp.concatenate(
        [jnp.tile(ao2[:, None], (1, 8)), jnp.zeros((DF, 8), jnp.float32)], axis=1)

    col = jnp.arange(DF)
    head = col // NHID
    feat = col % NHID
    pos = 32 * (head // 2) + 2 * feat + (head % 2)
    Pmat = jnp.zeros((DF, DF), jnp.float32).at[col, pos].set(1.0)

    zeros_blk = jnp.zeros((ZROWS, TW), jnp.float32)

    t1, s1a, s2a = _pre_call(x, wall, A1p, A2p, Pmat)
    acc1 = _edge_pass(t1, s1a, s2a, src, dst, zeros_blk)
    t2, s1b, s2b = _mid_call(acc1, W_out, A1o, A2o, Pmat)
    acc2 = _edge_pass(t2, s1b, s2b, src, dst, zeros_blk)
    return _post_call(acc2)


# speedup vs baseline: 1.0333x; 1.0333x over previous
import functools

import numpy as np

import jax
import jax.numpy as jnp
from jax import lax
from jax.experimental import pallas as pl
from jax.experimental.pallas import tpu as pltpu
from jax.experimental.pallas import tpu_sc as plsc

N = 10000
E = 320000
DF = 128
NHEADS = 8
NHID = 16
TW = 144
ALPHA = 0.2
EPS = 1e-16

NC = 2
NS = 16
EPT = E // (NC * NS)
CHUNK = 50
NCHUNK = EPT // CHUNK
IBLK = 50
NIBLK = NCHUNK // IBLK

_COL = np.arange(DF)
_HEAD = _COL // NHID
_FEAT = _COL % NHID
_HEAD_ONEHOT = np.zeros((DF, 16), np.float32)
_HEAD_ONEHOT[_COL, _HEAD] = 1.0
_PMAT = np.zeros((DF, DF), np.float32)
_PMAT[_COL, 32 * (_HEAD // 2) + 2 * _FEAT + (_HEAD % 2)] = 1.0
NPAD = 10240
ZROWS = NPAD // NS

ROWBLK = 1000
NBLK = N // ROWBLK


def _pre_body(x_ref, wall_ref, a1_ref, a2_ref, pmat_ref, tb_ref, s1_ref,
              s2_ref):
    h = jnp.dot(x_ref[...], wall_ref[...], preferred_element_type=jnp.float32)
    s1_ref[...] = jnp.dot(h, a1_ref[...], preferred_element_type=jnp.float32)
    s2_ref[...] = jnp.dot(h, a2_ref[...], preferred_element_type=jnp.float32)
    hp = jnp.dot(h, pmat_ref[...], preferred_element_type=jnp.float32)
    tb_ref[...] = hp.astype(jnp.bfloat16)


_pre_call = pl.pallas_call(
    _pre_body,
    grid=(NBLK,),
    in_specs=[
        pl.BlockSpec((ROWBLK, DF), lambda i: (i, 0)),
        pl.BlockSpec((DF, DF), lambda i: (0, 0)),
        pl.BlockSpec((DF, 16), lambda i: (0, 0)),
        pl.BlockSpec((DF, 16), lambda i: (0, 0)),
        pl.BlockSpec((DF, DF), lambda i: (0, 0)),
    ],
    out_specs=[
        pl.BlockSpec((ROWBLK, DF), lambda i: (i, 0)),
        pl.BlockSpec((ROWBLK, 16), lambda i: (i, 0)),
        pl.BlockSpec((ROWBLK, 16), lambda i: (i, 0)),
    ],
    out_shape=[
        jax.ShapeDtypeStruct((N, DF), jnp.bfloat16),
        jax.ShapeDtypeStruct((N, 16), jnp.float32),
        jax.ShapeDtypeStruct((N, 16), jnp.float32),
    ],
)


def _mid_body(acc_ref, wout_ref, a1_ref, a2_ref, pmat_ref, tb_ref, s1_ref,
              s2_ref):
    s = acc_ref[0] + acc_ref[1]
    cols = []
    for i in range(NHEADS):
        hi = s[:, NHID * i:NHID * (i + 1)]
        ri = s[:, DF + i:DF + i + 1]
        cols.append(hi / (ri + EPS))
    x2 = jnp.concatenate(cols, axis=1)
    x2 = jnp.where(x2 > 0, x2, jnp.exp(x2) - 1.0)
    h = jnp.dot(x2, wout_ref[...], preferred_element_type=jnp.float32)
    s1_ref[...] = jnp.dot(h, a1_ref[...], preferred_element_type=jnp.float32)
    s2_ref[...] = jnp.dot(h, a2_ref[...], preferred_element_type=jnp.float32)
    hp = jnp.dot(h, pmat_ref[...], preferred_element_type=jnp.float32)
    tb_ref[...] = hp.astype(jnp.bfloat16)


_mid_call = pl.pallas_call(
    _mid_body,
    grid=(NBLK,),
    in_specs=[
        pl.BlockSpec((NC, ROWBLK, TW), lambda i: (0, i, 0)),
        pl.BlockSpec((DF, DF), lambda i: (0, 0)),
        pl.BlockSpec((DF, 16), lambda i: (0, 0)),
        pl.BlockSpec((DF, 16), lambda i: (0, 0)),
        pl.BlockSpec((DF, DF), lambda i: (0, 0)),
    ],
    out_specs=[
        pl.BlockSpec((ROWBLK, DF), lambda i: (i, 0)),
        pl.BlockSpec((ROWBLK, 16), lambda i: (i, 0)),
        pl.BlockSpec((ROWBLK, 16), lambda i: (i, 0)),
    ],
    out_shape=[
        jax.ShapeDtypeStruct((N, DF), jnp.bfloat16),
        jax.ShapeDtypeStruct((N, 16), jnp.float32),
        jax.ShapeDtypeStruct((N, 16), jnp.float32),
    ],
)


def _post_body(acc_ref, o_ref):
    s = acc_ref[0] + acc_ref[1]
    x = s[:, 0:DF] / (s[:, DF:DF + 1] + EPS)
    o_ref[...] = jnp.where(x > 0, x, jnp.exp(x) - 1.0)


_post_call = pl.pallas_call(
    _post_body,
    grid=(NBLK,),
    in_specs=[pl.BlockSpec((NC, ROWBLK, TW), lambda i: (0, i, 0))],
    out_specs=pl.BlockSpec((ROWBLK, DF), lambda i: (i, 0)),
    out_shape=jax.ShapeDtypeStruct((N, DF), jnp.float32),
)


def _lane_broadcast(v, j):
    idx = jnp.full((16, 1), j, dtype=jnp.int32)
    dn = lax.GatherDimensionNumbers(
        offset_dims=(), collapsed_slice_dims=(0,), start_index_map=(0,))
    return lax.gather(v, idx, dn, slice_sizes=(1,),
                      mode=lax.GatherScatterMode.PROMISE_IN_BOUNDS)


def _edge_pass_body(t_hbm, s1_hbm, s2_hbm, src_hbm, dst_hbm, zeros_hbm,
                    out_hbm, srcblk, dstblk, gbuf0, gbuf1, s1b0, s1b1,
                    s2b0, s2b1, mb0, mb1, accum, gsem0, gsem1, s1sem0,
                    s1sem1, s2sem0, s2sem1, ssem0, ssem1):
    cid = lax.axis_index("c")
    sid = lax.axis_index("s")
    crow = (cid * NS + sid) * NCHUNK
    zrow0 = pl.multiple_of(sid * ZROWS, ZROWS)

    gbufs = [gbuf0, gbuf1]
    s1bufs = [s1b0, s1b1]
    s2bufs = [s2b0, s2b1]
    mbufs = [mb0, mb1]
    gsems = [gsem0, gsem1]
    s1sems = [s1sem0, s1sem1]
    s2sems = [s2sem0, s2sem1]
    ssems = [ssem0, ssem1]

    pltpu.sync_copy(zeros_hbm, accum.at[pl.ds(zrow0, ZROWS)])
    plsc.subcore_barrier()

    def issue_gather(c, b):
        pltpu.async_copy(t_hbm.at[dstblk.at[c]], gbufs[b], gsems[b])
        pltpu.async_copy(s1_hbm.at[srcblk.at[c]], s1bufs[b], s1sems[b])
        pltpu.async_copy(s2_hbm.at[dstblk.at[c]], s2bufs[b], s2sems[b])

    def wait_gather(c, b):
        pltpu.make_async_copy(t_hbm.at[dstblk.at[c]], gbufs[b], gsems[b]).wait()
        pltpu.make_async_copy(s1_hbm.at[srcblk.at[c]], s1bufs[b],
                              s1sems[b]).wait()
        pltpu.make_async_copy(s2_hbm.at[dstblk.at[c]], s2bufs[b],
                              s2sems[b]).wait()

    def issue_scatter(c, b):
        pltpu.async_copy(mbufs[b], accum.at[srcblk.at[c]], ssems[b], add=True)

    def wait_scatter(c, b):
        pltpu.make_async_copy(mbufs[b], accum.at[srcblk.at[c]],
                              ssems[b]).wait()

    def compute(b):
        gbuf, s1buf, s2buf, mbuf = gbufs[b], s1bufs[b], s2bufs[b], mbufs[b]

        def edge_body(e):
            s1 = s1buf[e, :]
            s2 = s2buf[e, :]
            t = s1 + s2
            w = jnp.exp(-jnp.maximum(t, ALPHA * t))
            mbuf[e, pl.ds(DF, 16)] = w
            for g in range(NHEADS // 2):
                hb = gbuf[e, pl.ds(32 * g, 32)]
                ha, hc = plsc.unpack(hb, format=plsc.PackFormat.INTERLEAVED,
                                     preferred_element_type=jnp.float32)
                mbuf[e, pl.ds(32 * g, 16)] = _lane_broadcast(w, 2 * g) * ha
                mbuf[e, pl.ds(32 * g + 16, 16)] = (
                    _lane_broadcast(w, 2 * g + 1) * hc)

        plsc.parallel_loop(0, CHUNK, 1, unroll=4)(edge_body)

    def block_body(blk, carry):
        brow = crow + blk * IBLK
        pltpu.sync_copy(src_hbm.at[pl.ds(brow, IBLK)], srcblk)
        pltpu.sync_copy(dst_hbm.at[pl.ds(brow, IBLK)], dstblk)
        issue_gather(0, 0)
        issue_gather(1, 1)

        def pair_body(p, carry2):
            for b in range(2):
                cb = 2 * p + b
                wait_gather(cb, b)

                @pl.when(cb >= 2)
                def _():
                    wait_scatter(cb - 2, b)

                compute(b)
                issue_scatter(cb, b)
                issue_gather(cb + 2, b)
            return carry2

        lax.fori_loop(0, IBLK // 2 - 1, pair_body, 0)

        for cb in (IBLK - 2, IBLK - 1):
            b = cb % 2
            wait_gather(cb, b)
            wait_scatter(cb - 2, b)
            compute(b)
            issue_scatter(cb, b)
        wait_scatter(IBLK - 2, (IBLK - 2) % 2)
        wait_scatter(IBLK - 1, (IBLK - 1) % 2)
        return carry

    lax.fori_loop(0, NIBLK, block_body, 0)

    plsc.subcore_barrier()
    pltpu.sync_copy(accum.at[pl.ds(zrow0, ZROWS)],
                    out_hbm.at[cid, pl.ds(zrow0, ZROWS)])


_EDGE_PASS_CACHE = []


def _edge_pass(*args):
    if not _EDGE_PASS_CACHE:
        mesh = plsc.VectorSubcoreMesh(core_axis_name="c", subcore_axis_name="s")
        _EDGE_PASS_CACHE.append(functools.partial(
            pl.kernel,
            out_type=jax.ShapeDtypeStruct((NC, NPAD, TW), jnp.float32),
            mesh=mesh,
            scratch_types=[
                pltpu.VMEM((IBLK, CHUNK), jnp.int32),
                pltpu.VMEM((IBLK, CHUNK), jnp.int32),
                pltpu.VMEM((CHUNK, DF), jnp.bfloat16),
                pltpu.VMEM((CHUNK, DF), jnp.bfloat16),
                pltpu.VMEM((CHUNK, 16), jnp.float32),
                pltpu.VMEM((CHUNK, 16), jnp.float32),
                pltpu.VMEM((CHUNK, 16), jnp.float32),
                pltpu.VMEM((CHUNK, 16), jnp.float32),
                pltpu.VMEM((CHUNK, TW), jnp.float32),
                pltpu.VMEM((CHUNK, TW), jnp.float32),
                pltpu.VMEM_SHARED((NPAD, TW), jnp.float32),
                pltpu.SemaphoreType.DMA,
                pltpu.SemaphoreType.DMA,
                pltpu.SemaphoreType.DMA,
                pltpu.SemaphoreType.DMA,
                pltpu.SemaphoreType.DMA,
                pltpu.SemaphoreType.DMA,
                pltpu.SemaphoreType.DMA,
                pltpu.SemaphoreType.DMA,
            ],
            compiler_params=pltpu.CompilerParams(
                use_tc_tiling_on_sc=False, needs_layout_passes=False),
        )(_edge_pass_body))
    return _EDGE_PASS_CACHE[0](*args)


def kernel(Corpus_, batch_inputs, entity_embeddings, edge_list, W, a, W_out, a_out):
    x = entity_embeddings
    src = edge_list[0].reshape(E // CHUNK, CHUNK)
    dst = edge_list[1].reshape(E // CHUNK, CHUNK)

    wall = jnp.transpose(W, (1, 0, 2)).reshape(DF, DF)
    a1 = a[:, 0, :NHID]
    a2 = a[:, 0, NHID:]
    onehot = jnp.asarray(_HEAD_ONEHOT)
    A1p = onehot * a1.reshape(-1)[:, None]
    A2p = onehot * a2.reshape(-1)[:, None]

    ao1 = a_out[0, :DF]
    ao2 = a_out[0, DF:]
    A1o = jnp.tile(ao1[:, None], (1, 16))
    A2o = jnp.concatenate(
        [jnp.tile(ao2[:, None], (1, 8)), jnp.zeros((DF, 8), jnp.float32)], axis=1)

    Pmat = jnp.asarray(_PMAT)
    zeros_blk = jnp.zeros((ZROWS, TW), jnp.float32)

    t1, s1a, s2a = _pre_call(x, wall, A1p, A2p, Pmat)
    acc1 = _edge_pass(t1, s1a, s2a, src, dst, zeros_blk)
    t2, s1b, s2b = _mid_call(acc1, W_out, A1o, A2o, Pmat)
    acc2 = _edge_pass(t2, s1b, s2b, src, dst, zeros_blk)
    return _post_call(acc2)
